# Initial kernel scaffold; baseline (speedup 1.0000x reference)
import jax
import jax.numpy as jnp
from jax.experimental import pallas as pl


def kernel(x, Wproj, bproj, Wqkv, bqkv, Wo, bo, g1, be1, gateW, W1, b1, W2, b2, g2, be2, gc, bc, Wc1, bc1, Wc2, bc2):
    raise NotImplementedError("write your pallas kernel here")


# speedup vs baseline: 2.7493x; 2.7493x over previous
import math
import functools

import jax
import jax.numpy as jnp
from jax.experimental import pallas as pl
from jax.experimental.pallas import tpu as pltpu

B, S, IN = 1, 2048, 32
D, H, DH = 768, 12, 64
E, DFF = 8, 3072
SCALE = 1.0 / math.sqrt(DH)


def _gelu(x):
    return 0.5 * x * (1.0 + jax.lax.erf(x * 0.7071067811865476))


def _ln(x, g, b):
    m = jnp.mean(x, axis=-1, keepdims=True)
    v = jnp.mean((x - m) ** 2, axis=-1, keepdims=True)
    return (x - m) * jax.lax.rsqrt(v + 1e-5) * g + b


def _k1a_body(x_ref, pe_ref, wproj_ref, bproj_ref, wqkv_ref, bqkv_ref,
              h_ref, q_ref, k_ref, v_ref):
    x = x_ref[...]
    h = jax.lax.dot_general(x, wproj_ref[...], (((1,), (1,)), ((), ())))
    h = h + bproj_ref[...] + pe_ref[...]
    h_ref[...] = h
    for hh in range(H):
        for off, ref in ((0, q_ref), (D, k_ref), (2 * D, v_ref)):
            w = wqkv_ref[off + hh * DH:off + (hh + 1) * DH, :]
            b = bqkv_ref[:, off + hh * DH:off + (hh + 1) * DH]
            ref[hh] = jax.lax.dot_general(
                h, w, (((1,), (1,)), ((), ()))) + b


def _k1b_body(q_ref, k_ref, v_ref, ao_ref):
    s = jax.lax.dot_general(q_ref[0], k_ref[0],
                            (((1,), (1,)), ((), ()))) * SCALE
    p = jax.nn.softmax(s, axis=-1)
    ao_ref[0] = jnp.dot(p, v_ref[0])


def _k1c_body(h_ref, ao_ref, wo_ref, bo_ref, g1_ref, be1_ref, gatew_ref,
              h1_ref, cw_ref):
    ao = bo_ref[...]
    for hh in range(H):
        ao = ao + jax.lax.dot_general(
            ao_ref[hh], wo_ref[:, hh * DH:(hh + 1) * DH],
            (((1,), (1,)), ((), ())))
    h1 = _ln(h_ref[...] + ao, g1_ref[...], be1_ref[...])
    h1_ref[...] = h1
    logits = jax.lax.dot_general(h1, gatew_ref[...], (((1,), (1,)), ((), ())))
    probs = jax.nn.softmax(logits, axis=-1)
    lane = jax.lax.broadcasted_iota(jnp.int32, (S, E), 1)
    m1 = jnp.max(probs, axis=-1, keepdims=True)
    i1 = jnp.min(jnp.where(probs == m1, lane, E), axis=-1, keepdims=True)
    probs2 = jnp.where(lane == i1, -1.0, probs)
    m2 = jnp.max(probs2, axis=-1, keepdims=True)
    i2 = jnp.min(jnp.where(probs2 == m2, lane, E), axis=-1, keepdims=True)
    denom = m1 + m2 + 1e-8
    lane_p = jax.lax.broadcasted_iota(jnp.int32, (S, 128), 1)
    cw = jnp.where(lane_p == i1, m1 / denom, 0.0)
    cw = jnp.where(lane_p == i2, m2 / denom, cw)
    cw_ref[...] = cw


def _k2_body(x_ref, w1_ref, b1_ref, w2_ref, b2_ref, cw_ref, out_ref):
    e = pl.program_id(0)
    c = pl.program_id(1)
    x = x_ref[...]
    w1 = w1_ref[0]
    hid = jax.lax.dot_general(x, w1, (((1,), (1,)), ((), ())))
    hid = _gelu(hid + b1_ref[0])
    w2 = w2_ref[0]
    ye = jax.lax.dot_general(hid, w2, (((1,), (1,)), ((), ())))
    lane = jax.lax.broadcasted_iota(jnp.int32, (S, 128), 1)
    col = jnp.sum(jnp.where(lane == e, cw_ref[...], 0.0), axis=-1,
                  keepdims=True)
    contrib = col * (ye + jnp.where(c == 0, 1.0, 0.0) * b2_ref[0])

    @pl.when((e == 0) & (c == 0))
    def _():
        out_ref[...] = contrib

    @pl.when((e != 0) | (c != 0))
    def _():
        out_ref[...] = out_ref[...] + contrib


def _k3_body(h1_ref, moe_ref, g2_ref, be2_ref, gc_ref, bc_ref,
             wc1_ref, bc1_ref, wc2_ref, bc2_ref, out_ref):
    h2 = _ln(h1_ref[...] + moe_ref[...], g2_ref[...], be2_ref[...])
    pooled = jnp.mean(h2, axis=0, keepdims=True)
    c = _ln(pooled, gc_ref[...], bc_ref[...])
    c = jax.lax.dot_general(c, wc1_ref[...], (((1,), (1,)), ((), ())))
    c = _gelu(c + bc1_ref[...])
    out_ref[...] = jnp.sum(c * wc2_ref[...], axis=-1, keepdims=True) + bc2_ref[...]


def _pe_table():
    position = jnp.arange(S, dtype=jnp.float32)[:, None]
    div_term = jnp.exp(jnp.arange(0, D, 2, dtype=jnp.float32)
                       * (-math.log(10000.0) / D))
    pe = jnp.zeros((S, D), dtype=jnp.float32)
    pe = pe.at[:, 0::2].set(jnp.sin(position * div_term))
    pe = pe.at[:, 1::2].set(jnp.cos(position * div_term))
    return pe


def kernel(x, Wproj, bproj, Wqkv, bqkv, Wo, bo, g1, be1, gateW, W1, b1,
           W2, b2, g2, be2, gc, bc, Wc1, bc1, Wc2, bc2):
    xr = x.reshape(S, IN)
    pe = _pe_table()
    f32 = jnp.float32
    h, q3, k3, v3 = pl.pallas_call(
        _k1a_body,
        out_shape=[jax.ShapeDtypeStruct((S, D), f32),
                   jax.ShapeDtypeStruct((H, S, DH), f32),
                   jax.ShapeDtypeStruct((H, S, DH), f32),
                   jax.ShapeDtypeStruct((H, S, DH), f32)],
    )(xr, pe, Wproj, bproj.reshape(1, D), Wqkv, bqkv.reshape(1, 3 * D))

    ao = pl.pallas_call(
        _k1b_body,
        grid=(H,),
        in_specs=[
            pl.BlockSpec((1, S, DH), lambda hh: (hh, 0, 0)),
            pl.BlockSpec((1, S, DH), lambda hh: (hh, 0, 0)),
            pl.BlockSpec((1, S, DH), lambda hh: (hh, 0, 0)),
        ],
        out_specs=pl.BlockSpec((1, S, DH), lambda hh: (hh, 0, 0)),
        out_shape=jax.ShapeDtypeStruct((H, S, DH), f32),
        compiler_params=pltpu.CompilerParams(
            dimension_semantics=("parallel",)),
    )(q3, k3, v3)

    h1, cw = pl.pallas_call(
        _k1c_body,
        out_shape=[jax.ShapeDtypeStruct((S, D), f32),
                   jax.ShapeDtypeStruct((S, 128), f32)],
    )(h, ao, Wo, bo.reshape(1, D), g1.reshape(1, D), be1.reshape(1, D), gateW)

    NC = 2
    DFFC = DFF // NC
    moe = pl.pallas_call(
        _k2_body,
        grid=(E, NC),
        in_specs=[
            pl.BlockSpec((S, D), lambda e, c: (0, 0)),
            pl.BlockSpec((1, DFFC, D), lambda e, c: (e, c, 0)),
            pl.BlockSpec((1, 1, DFFC), lambda e, c: (e, 0, c)),
            pl.BlockSpec((1, D, DFFC), lambda e, c: (e, 0, c)),
            pl.BlockSpec((1, 1, D), lambda e, c: (e, 0, 0)),
            pl.BlockSpec((S, 128), lambda e, c: (0, 0)),
        ],
        out_specs=pl.BlockSpec((S, D), lambda e, c: (0, 0)),
        out_shape=jax.ShapeDtypeStruct((S, D), f32),
        compiler_params=pltpu.CompilerParams(
            dimension_semantics=("arbitrary", "arbitrary")),
    )(h1, W1, b1.reshape(E, 1, DFF), W2, b2.reshape(E, 1, D), cw)

    out = pl.pallas_call(
        _k3_body,
        out_shape=jax.ShapeDtypeStruct((1, 1), f32),
    )(h1, moe, g2.reshape(1, D), be2.reshape(1, D), gc.reshape(1, D),
      bc.reshape(1, D), Wc1, bc1.reshape(1, D // 2), Wc2,
      bc2.reshape(1, 1))
    return out.reshape(B)
